# baseline (speedup 1.0000x reference)
import functools

import jax
import jax.numpy as jnp
from jax import lax
from jax.experimental import pallas as pl
from jax.experimental.pallas import tpu as pltpu
from jax.experimental.pallas import tpu_sc as plsc

BATCH = 16384
D = 64
H1 = 128
H2 = 64
NC = 2
NS = 16
NW = NC * NS
BPW = BATCH // NW

@functools.cache
def _make_gather_embeddings():
    mesh = plsc.VectorSubcoreMesh(core_axis_name="c", subcore_axis_name="s")

    @functools.partial(
        pl.kernel,
        out_type=(
            jax.ShapeDtypeStruct((BATCH, D), jnp.float32),
            jax.ShapeDtypeStruct((BATCH, D), jnp.float32),
        ),
        mesh=mesh,
        scratch_types=[
            pltpu.VMEM((BPW,), jnp.int32),
            pltpu.VMEM((BPW, D), jnp.float32),
            pltpu.VMEM((BPW,), jnp.int32),
            pltpu.VMEM((BPW, D), jnp.float32),
            pltpu.SemaphoreType.DMA,
            pltpu.SemaphoreType.DMA,
        ],
        compiler_params=pltpu.CompilerParams(use_tc_tiling_on_sc=False),
    )
    def gather_embeddings(uid_hbm, bid_hbm, utab_hbm, btab_hbm,
                          uout_hbm, bout_hbm,
                          uidx_v, urows_v, bidx_v, brows_v, usem, bsem):
        wid = lax.axis_index("s") * NC + lax.axis_index("c")
        base = wid * BPW
        pltpu.sync_copy(uid_hbm.at[pl.ds(base, BPW)], uidx_v)
        pltpu.sync_copy(bid_hbm.at[pl.ds(base, BPW)], bidx_v)
        cu = pltpu.async_copy(utab_hbm.at[uidx_v], urows_v, usem)
        cb = pltpu.async_copy(btab_hbm.at[bidx_v], brows_v, bsem)
        cu.wait()
        cb.wait()
        pltpu.sync_copy(urows_v, uout_hbm.at[pl.ds(base, BPW)])
        pltpu.sync_copy(brows_v, bout_hbm.at[pl.ds(base, BPW)])

    return gather_embeddings


BT = 1024


def _mlp_body(u_ref, b_ref, w1u_ref, w1b_ref, b1_ref, w2_ref, b2_ref,
              w3_ref, b3_ref, out_ref):
    x1 = jnp.dot(u_ref[...], w1u_ref[...], preferred_element_type=jnp.float32)
    x1 += jnp.dot(b_ref[...], w1b_ref[...], preferred_element_type=jnp.float32)
    x1 = jnp.maximum(x1 + b1_ref[...], 0.0)
    x2 = jnp.dot(x1, w2_ref[...], preferred_element_type=jnp.float32)
    x2 = jnp.maximum(x2 + b2_ref[...], 0.0)
    out_ref[...] = jnp.sum(x2 * w3_ref[...], axis=1) + b3_ref[0, 0]


def kernel(user_id, book_id, user_table, book_table, W1, b1, W2, b2, W3, b3):
    uemb, bemb = _make_gather_embeddings()(user_id, book_id, user_table,
                                           book_table)
    w1t = W1.T
    out = pl.pallas_call(
        _mlp_body,
        grid=(BATCH // BT,),
        in_specs=[
            pl.BlockSpec((BT, D), lambda i: (i, 0)),
            pl.BlockSpec((BT, D), lambda i: (i, 0)),
            pl.BlockSpec((D, H1), lambda i: (0, 0)),
            pl.BlockSpec((D, H1), lambda i: (0, 0)),
            pl.BlockSpec((1, H1), lambda i: (0, 0)),
            pl.BlockSpec((H1, H2), lambda i: (0, 0)),
            pl.BlockSpec((1, H2), lambda i: (0, 0)),
            pl.BlockSpec((1, H2), lambda i: (0, 0)),
            pl.BlockSpec((1, 1), lambda i: (0, 0)),
        ],
        out_specs=pl.BlockSpec((BT,), lambda i: (i,)),
        out_shape=jax.ShapeDtypeStruct((BATCH,), jnp.float32),
    )(uemb, bemb, w1t[:D], w1t[D:], b1.reshape(1, H1), W2.T,
      b2.reshape(1, H2), W3, b3.reshape(1, 1))
    return out
